# baseline (speedup 1.0000x reference)
import functools

import jax
import jax.numpy as jnp
from jax.experimental import pallas as pl

WINDOW = 16
NUM_BINS = 100
SCALE = 0.1
EPS = 1e-08
INV_H = (NUM_BINS - 2) / (2.0 * SCALE)
MID = (NUM_BINS - 2) // 2


def _tok_kernel(vals_ref, bins_ref, last_ref, mean_ref, std_ref):
    bb, tc, s = vals_ref.shape
    x = vals_ref[...].reshape(bb * tc, s)
    nw = (bb * tc) // WINDOW
    w = x.reshape(nw, WINDOW, s)

    last_ref[...] = w[:, WINDOW - 1, :].reshape(bb, tc // WINDOW, s)
    mean = jnp.mean(w, axis=1)
    mean_ref[...] = mean.reshape(bb, tc // WINDOW, s)
    centered = w - mean[:, None, :]
    std = jnp.sqrt(jnp.mean(centered * centered, axis=1)) + EPS
    std_ref[...] = std.reshape(bb, tc // WINDOW, s)

    nxt = jnp.concatenate([x[1:], x[-1:]], axis=0)
    delta = (nxt - x) / jnp.maximum(jnp.abs(x), EPS)
    u = delta * INV_H + float(MID)
    b = jnp.clip(jnp.ceil(u), 0.0, float(NUM_BINS - 1)).astype(jnp.int32)
    bins_ref[...] = b.reshape(bb, tc, s)


@functools.partial(jax.jit, static_argnames=("tc", "bb"))
def _run(values, tc=4096, bb=2):
    bsz, t, s = values.shape
    nw_total = t // WINDOW
    nw = tc // WINDOW
    stat = pl.BlockSpec((bb, nw, s), lambda b: (b, 0, 0))
    stat_shape = jax.ShapeDtypeStruct((bsz, nw_total, s), jnp.float32)
    bins, last, mean, std = pl.pallas_call(
        _tok_kernel,
        grid=(bsz // bb,),
        in_specs=[pl.BlockSpec((bb, tc, s), lambda b: (b, 0, 0))],
        out_specs=[
            pl.BlockSpec((bb, tc, s), lambda b: (b, 0, 0)),
            stat, stat, stat,
        ],
        out_shape=[
            jax.ShapeDtypeStruct((bsz, t, s), jnp.int32),
            stat_shape, stat_shape, stat_shape,
        ],
    )(values)
    return bins, last, mean, std


def _finish(values_half, bins, last, mean, std):
    bsz, t, s = values_half.shape
    nw_total = t // WINDOW
    lf = jnp.stack([last, mean, std], axis=-1)
    bins = bins.reshape(bsz, nw_total, WINDOW, s)[:, :, : WINDOW - 1, :]
    bins = jnp.swapaxes(bins, 2, 3)
    return bins, lf


def kernel(values):
    half = values.shape[0] // 2
    h1, h2 = values[:half], values[half:]
    b1, lf1 = _finish(h1, *_run(h1))
    b2, lf2 = _finish(h2, *_run(h2))
    bins = jnp.concatenate([b1, b2], axis=0)
    lf = jnp.concatenate([lf1, lf2], axis=0)
    return bins.astype(jnp.int64), lf


# speedup vs baseline: 1.1977x; 1.1977x over previous
import functools

import jax
import jax.numpy as jnp
from jax.experimental import pallas as pl

WINDOW = 16
NUM_BINS = 100
SCALE = 0.1
EPS = 1e-08
INV_H = (NUM_BINS - 2) / (2.0 * SCALE)
MID = (NUM_BINS - 2) // 2


def _tok_kernel(vals_ref, bins_ref, last_ref, mean_ref, std_ref):
    bb, tc, s = vals_ref.shape
    x = vals_ref[...].reshape(bb * tc, s)
    nw = (bb * tc) // WINDOW
    w = x.reshape(nw, WINDOW, s)

    last_ref[...] = w[:, WINDOW - 1, :].reshape(bb, tc // WINDOW, s)
    mean = jnp.mean(w, axis=1)
    mean_ref[...] = mean.reshape(bb, tc // WINDOW, s)
    centered = w - mean[:, None, :]
    std = jnp.sqrt(jnp.mean(centered * centered, axis=1)) + EPS
    std_ref[...] = std.reshape(bb, tc // WINDOW, s)

    nxt = jnp.concatenate([x[1:], x[-1:]], axis=0)
    delta = (nxt - x) / jnp.maximum(jnp.abs(x), EPS)
    u = delta * INV_H + float(MID)
    b = jnp.clip(jnp.ceil(u), 0.0, float(NUM_BINS - 1)).astype(jnp.int32)
    bins_ref[...] = b.reshape(bb, tc, s)


@functools.partial(jax.jit, static_argnames=("tc", "bb"))
def _run(values, tc=4096, bb=2):
    bsz, t, s = values.shape
    nw_total = t // WINDOW
    nw = tc // WINDOW
    stat = pl.BlockSpec((bb, nw, s), lambda b: (b, 0, 0))
    stat_shape = jax.ShapeDtypeStruct((bsz, nw_total, s), jnp.float32)
    bins, last, mean, std = pl.pallas_call(
        _tok_kernel,
        grid=(bsz // bb,),
        in_specs=[pl.BlockSpec((bb, tc, s), lambda b: (b, 0, 0))],
        out_specs=[
            pl.BlockSpec((bb, tc, s), lambda b: (b, 0, 0)),
            stat, stat, stat,
        ],
        out_shape=[
            jax.ShapeDtypeStruct((bsz, t, s), jnp.int32),
            stat_shape, stat_shape, stat_shape,
        ],
    )(values)
    return bins, last, mean, std


def kernel(values):
    bins, last, mean, std = _run(values)
    bsz, t, s = values.shape
    nw_total = t // WINDOW
    lf = jnp.stack([last, mean, std], axis=-1)
    bins = bins.reshape(bsz, nw_total, WINDOW, s)[:, :, : WINDOW - 1, :]
    bins = jnp.swapaxes(bins, 2, 3)
    return bins.astype(jnp.int64), lf


# speedup vs baseline: 1.2481x; 1.0420x over previous
import functools

import jax
import jax.numpy as jnp
from jax.experimental import pallas as pl

WINDOW = 16
NUM_BINS = 100
SCALE = 0.1
EPS = 1e-08
INV_H = (NUM_BINS - 2) / (2.0 * SCALE)
MID = (NUM_BINS - 2) // 2


def _tok_kernel(vals_ref, bins_ref, last_ref, mean_ref, std_ref):
    bb, tc, s = vals_ref.shape
    x = vals_ref[...].reshape(bb * tc, s)
    nw = (bb * tc) // WINDOW
    w = x.reshape(nw, WINDOW, s)

    last_ref[...] = w[:, WINDOW - 1, :].reshape(bb, tc // WINDOW, s)
    mean = jnp.mean(w, axis=1)
    mean_ref[...] = mean.reshape(bb, tc // WINDOW, s)
    centered = w - mean[:, None, :]
    std = jnp.sqrt(jnp.mean(centered * centered, axis=1)) + EPS
    std_ref[...] = std.reshape(bb, tc // WINDOW, s)

    nxt = jnp.concatenate([x[1:], x[-1:]], axis=0)
    delta = (nxt - x) / jnp.maximum(jnp.abs(x), EPS)
    u = delta * INV_H + float(MID)
    b = jnp.clip(jnp.ceil(u), 0.0, float(NUM_BINS - 1)).astype(jnp.int8)
    bins_ref[...] = b.reshape(bb, tc, s)


@functools.partial(jax.jit, static_argnames=("tc", "bb"))
def _run(values, tc=4096, bb=2):
    bsz, t, s = values.shape
    nw_total = t // WINDOW
    nw = tc // WINDOW
    stat = pl.BlockSpec((bb, nw, s), lambda b: (b, 0, 0))
    stat_shape = jax.ShapeDtypeStruct((bsz, nw_total, s), jnp.float32)
    bins, last, mean, std = pl.pallas_call(
        _tok_kernel,
        grid=(bsz // bb,),
        in_specs=[pl.BlockSpec((bb, tc, s), lambda b: (b, 0, 0))],
        out_specs=[
            pl.BlockSpec((bb, tc, s), lambda b: (b, 0, 0)),
            stat, stat, stat,
        ],
        out_shape=[
            jax.ShapeDtypeStruct((bsz, t, s), jnp.int8),
            stat_shape, stat_shape, stat_shape,
        ],
    )(values)
    return bins, last, mean, std


def kernel(values):
    bins, last, mean, std = _run(values)
    bsz, t, s = values.shape
    nw_total = t // WINDOW
    lf = jnp.stack([last, mean, std], axis=-1)
    bins = bins.reshape(bsz, nw_total, WINDOW, s)[:, :, : WINDOW - 1, :]
    bins = jnp.swapaxes(bins, 2, 3)
    return bins.astype(jnp.int64), lf
